# baseline (speedup 1.0000x reference)
import jax
import jax.numpy as jnp
from jax import lax
from jax.experimental import pallas as pl
from jax.experimental.pallas import tpu as pltpu
from jax.experimental.pallas import tpu_sc as plsc

NUM_EMB = 100000
DIM = 64
OUT_W = 128
BAND_LO = 32000
N_TRAIN = 16
BATCH = 4096
HIST = 200
TOTAL = BATCH * HIST
NB = 2
CHUNK = NB * HIST
LANES = 16
SPLIT = 128
REST = HIST - SPLIT
X_W = 256
W_W = 128
N_FULL = HIST // LANES
TAIL_OFF = HIST - LANES
TAIL_LO = N_FULL * LANES - TAIL_OFF


def _sc_body(x_hbm, w_hbm, delta_hbm, out_hbm,
             idx0, idx1, rows0, rows1, dflat_v,
             gsem0, gsem1, wsem, isem):
    info = plsc.get_sparse_core_info()
    nc = info.num_cores
    nw = nc * info.num_subcores
    rows_per_w = BATCH // nw
    n_chunks = rows_per_w // NB

    idxb = (idx0, idx1)
    rowsb = (rows0, rows1)
    gsem = (gsem0, gsem1)

    wid = lax.axis_index("s") * nc + lax.axis_index("c")
    row_base = wid * rows_per_w

    pltpu.sync_copy(delta_hbm, dflat_v)

    lane = lax.iota(jnp.int32, LANES)

    def issue_idx(c, p):
        pltpu.async_copy(
            x_hbm.at[pl.ds(row_base + c * NB, NB)], idxb[p], isem)

    def drain_idx(p):
        pltpu.make_async_copy(
            x_hbm.at[pl.ds(0, NB)], idxb[p], isem).wait()

    def issue_gathers(p):
        for r in range(NB):
            pltpu.async_copy(
                w_hbm.at[idxb[p].at[r, pl.ds(0, SPLIT)]],
                rowsb[p].at[pl.ds(r * HIST, SPLIT)],
                gsem[p])
            pltpu.async_copy(
                w_hbm.at[idxb[p].at[r, pl.ds(SPLIT, REST)]],
                rowsb[p].at[pl.ds(r * HIST + SPLIT, REST)],
                gsem[p])

    def drain_gathers(p):
        for r in range(NB):
            pltpu.make_async_copy(
                w_hbm.at[idxb[p].at[r, pl.ds(0, SPLIT)]],
                rowsb[p].at[pl.ds(r * HIST, SPLIT)],
                gsem[p]).wait()
            pltpu.make_async_copy(
                w_hbm.at[idxb[p].at[r, pl.ds(SPLIT, REST)]],
                rowsb[p].at[pl.ds(r * HIST + SPLIT, REST)],
                gsem[p]).wait()

    def issue_write(c, p):
        pltpu.async_copy(
            rowsb[p].at[pl.ds(0, CHUNK), pl.ds(0, DIM)],
            out_hbm.at[pl.ds((row_base + c * NB) * HIST, CHUNK),
                       pl.ds(0, DIM)],
            wsem)

    def drain_write(p):
        pltpu.make_async_copy(
            rowsb[p].at[pl.ds(0, CHUNK), pl.ds(0, DIM)],
            out_hbm.at[pl.ds(0, CHUNK), pl.ds(0, DIM)],
            wsem).wait()

    def fixup(p):
        def fix_row(r, _):
            def fix_at(off, base_mask):
                idxs = idxb[p][r, pl.ds(off, LANES)]
                rel = idxs - BAND_LO
                m = (rel >= 0) & (rel < N_TRAIN) & base_mask

                @pl.when(jnp.any(m))
                def _():
                    relc = jnp.where(m, rel, 0)
                    pos = r * HIST + off + lane

                    def fix_col(col, _):
                        dvals = plsc.load_gather(
                            dflat_v, [col * N_TRAIN + relc], mask=m)
                        plsc.addupdate_scatter(
                            rowsb[p],
                            [pos, jnp.full((LANES,), 1, jnp.int32) * col],
                            dvals,
                            mask=m,
                        )
                        return _

                    lax.fori_loop(0, DIM, fix_col, None, unroll=False)

            def fix_slice(s, _):
                fix_at(s * LANES, lane >= 0)
                return _

            lax.fori_loop(0, N_FULL, fix_slice, None, unroll=False)
            fix_at(TAIL_OFF, lane >= TAIL_LO)
            return _

        lax.fori_loop(0, NB, fix_row, None, unroll=False)

    pltpu.sync_copy(x_hbm.at[pl.ds(row_base, NB)], idx0)
    issue_gathers(0)
    issue_idx(1, 1)

    def pair_body(g, _):
        for p in range(2):
            c = g * 2 + p
            q = 1 - p

            @pl.when(c < n_chunks - 1)
            def _():
                drain_idx(q)

            @pl.when(c >= 1)
            def _():
                drain_write(q)

            @pl.when(c < n_chunks - 1)
            def _():
                issue_gathers(q)

            drain_gathers(p)
            fixup(p)

            @pl.when(c < n_chunks - 2)
            def _():
                issue_idx(c + 2, p)

            issue_write(c, p)
        return _

    lax.fori_loop(0, n_chunks // 2, pair_body, None, unroll=False)
    drain_write((n_chunks - 1) % 2)


def kernel(x, W, delta):
    x_i = x if x.dtype == jnp.int32 else x.astype(jnp.int32)
    x_pad = jnp.pad(x_i, ((0, 0), (0, X_W - HIST)))
    w_pad = jnp.pad(W, ((0, 0), (0, W_W - DIM)))
    mesh = plsc.VectorSubcoreMesh(core_axis_name="c", subcore_axis_name="s")
    run = pl.kernel(
        _sc_body,
        out_type=jax.ShapeDtypeStruct((TOTAL, OUT_W), jnp.float32),
        mesh=mesh,
        scratch_types=[
            pltpu.VMEM((NB, X_W), jnp.int32),
            pltpu.VMEM((NB, X_W), jnp.int32),
            pltpu.VMEM((CHUNK, W_W), jnp.float32),
            pltpu.VMEM((CHUNK, W_W), jnp.float32),
            pltpu.VMEM((N_TRAIN * DIM,), jnp.float32),
            pltpu.SemaphoreType.DMA,
            pltpu.SemaphoreType.DMA,
            pltpu.SemaphoreType.DMA,
            pltpu.SemaphoreType.DMA,
        ],
        compiler_params=pltpu.CompilerParams(
            needs_layout_passes=False, use_tc_tiling_on_sc=False),
    )
    out = run(x_pad, w_pad, delta)
    return out.reshape(BATCH, HIST, OUT_W)[:, :, :DIM]


# speedup vs baseline: 1.1616x; 1.1616x over previous
import jax
import jax.numpy as jnp
from jax import lax
from jax.experimental import pallas as pl
from jax.experimental.pallas import tpu as pltpu
from jax.experimental.pallas import tpu_sc as plsc

NUM_EMB = 100000
DIM = 64
OUT_W = 128
BAND_LO = 32000
N_TRAIN = 16
BATCH = 4096
HIST = 200
TOTAL = BATCH * HIST
NB = 4
CHUNK = NB * HIST
LANES = 16
SPLIT = 128
REST = HIST - SPLIT
N_FULL = HIST // LANES
TAIL_OFF = HIST - LANES
TAIL_LO = N_FULL * LANES - TAIL_OFF


def _sc_body(x_hbm, w_hbm, delta_hbm, out_hbm,
             idx0, idx1, rows0, rows1, dflat_v,
             gsem0, gsem1, wsem, isem):
    info = plsc.get_sparse_core_info()
    nc = info.num_cores
    nw = nc * info.num_subcores
    rows_per_w = BATCH // nw
    n_chunks = rows_per_w // NB

    idxb = (idx0, idx1)
    rowsb = (rows0, rows1)
    gsem = (gsem0, gsem1)

    wid = lax.axis_index("s") * nc + lax.axis_index("c")
    row_base = wid * rows_per_w

    pltpu.sync_copy(delta_hbm, dflat_v)

    lane = lax.iota(jnp.int32, LANES)

    def issue_idx(c, p):
        pltpu.async_copy(
            x_hbm.at[pl.ds(row_base + c * NB, NB)], idxb[p], isem)

    def drain_idx(p):
        pltpu.make_async_copy(
            x_hbm.at[pl.ds(0, NB)], idxb[p], isem).wait()

    def issue_gathers(p):
        for r in range(NB):
            pltpu.async_copy(
                w_hbm.at[idxb[p].at[r, pl.ds(0, SPLIT)]],
                rowsb[p].at[pl.ds(r * HIST, SPLIT)],
                gsem[p])
            pltpu.async_copy(
                w_hbm.at[idxb[p].at[r, pl.ds(SPLIT, REST)]],
                rowsb[p].at[pl.ds(r * HIST + SPLIT, REST)],
                gsem[p])

    def drain_gathers(p):
        for r in range(NB):
            pltpu.make_async_copy(
                w_hbm.at[idxb[p].at[r, pl.ds(0, SPLIT)]],
                rowsb[p].at[pl.ds(r * HIST, SPLIT)],
                gsem[p]).wait()
            pltpu.make_async_copy(
                w_hbm.at[idxb[p].at[r, pl.ds(SPLIT, REST)]],
                rowsb[p].at[pl.ds(r * HIST + SPLIT, REST)],
                gsem[p]).wait()

    def issue_write(c, p):
        pltpu.async_copy(
            rowsb[p],
            out_hbm.at[pl.ds((row_base + c * NB) * HIST, CHUNK),
                       pl.ds(0, DIM)],
            wsem)

    def drain_write(p):
        pltpu.make_async_copy(
            rowsb[p],
            out_hbm.at[pl.ds(0, CHUNK), pl.ds(0, DIM)],
            wsem).wait()

    def fixup(p):
        def fix_row(r, _):
            def fix_at(off, base_mask):
                idxs = idxb[p][r, pl.ds(off, LANES)]
                rel = idxs - BAND_LO
                m = (rel >= 0) & (rel < N_TRAIN) & base_mask

                @pl.when(jnp.any(m))
                def _():
                    relc = jnp.where(m, rel, 0)
                    pos = r * HIST + off + lane

                    def fix_col(col, _):
                        dvals = plsc.load_gather(
                            dflat_v, [col * N_TRAIN + relc], mask=m)
                        plsc.addupdate_scatter(
                            rowsb[p],
                            [pos, jnp.full((LANES,), 1, jnp.int32) * col],
                            dvals,
                            mask=m,
                        )
                        return _

                    lax.fori_loop(0, DIM, fix_col, None, unroll=False)

            def fix_slice(s, _):
                fix_at(s * LANES, lane >= 0)
                return _

            lax.fori_loop(0, N_FULL, fix_slice, None, unroll=False)
            fix_at(TAIL_OFF, lane >= TAIL_LO)
            return _

        lax.fori_loop(0, NB, fix_row, None, unroll=False)

    pltpu.sync_copy(x_hbm.at[pl.ds(row_base, NB)], idx0)
    issue_gathers(0)
    issue_idx(1, 1)

    def pair_body(g, _):
        for p in range(2):
            c = g * 2 + p
            q = 1 - p

            @pl.when(c < n_chunks - 1)
            def _():
                drain_idx(q)

            @pl.when(c >= 1)
            def _():
                drain_write(q)

            @pl.when(c < n_chunks - 1)
            def _():
                issue_gathers(q)

            drain_gathers(p)
            fixup(p)

            @pl.when(c < n_chunks - 2)
            def _():
                issue_idx(c + 2, p)

            issue_write(c, p)
        return _

    lax.fori_loop(0, n_chunks // 2, pair_body, None, unroll=False)
    drain_write((n_chunks - 1) % 2)


def kernel(x, W, delta):
    x_i = x if x.dtype == jnp.int32 else x.astype(jnp.int32)
    mesh = plsc.VectorSubcoreMesh(core_axis_name="c", subcore_axis_name="s")
    run = pl.kernel(
        _sc_body,
        out_type=jax.ShapeDtypeStruct((TOTAL, OUT_W), jnp.float32),
        mesh=mesh,
        scratch_types=[
            pltpu.VMEM((NB, HIST), jnp.int32),
            pltpu.VMEM((NB, HIST), jnp.int32),
            pltpu.VMEM((CHUNK, DIM), jnp.float32),
            pltpu.VMEM((CHUNK, DIM), jnp.float32),
            pltpu.VMEM((N_TRAIN * DIM,), jnp.float32),
            pltpu.SemaphoreType.DMA,
            pltpu.SemaphoreType.DMA,
            pltpu.SemaphoreType.DMA,
            pltpu.SemaphoreType.DMA,
        ],
        compiler_params=pltpu.CompilerParams(
            needs_layout_passes=False, use_tc_tiling_on_sc=False),
    )
    out = run(x_i, W, delta)
    return out.reshape(BATCH, HIST, OUT_W)[:, :, :DIM]


# speedup vs baseline: 1.1675x; 1.0051x over previous
import jax
import jax.numpy as jnp
from jax import lax
from jax.experimental import pallas as pl
from jax.experimental.pallas import tpu as pltpu
from jax.experimental.pallas import tpu_sc as plsc

NUM_EMB = 100000
DIM = 64
OUT_W = 128
BAND_LO = 32000
N_TRAIN = 16
BATCH = 4096
HIST = 200
TOTAL = BATCH * HIST
XCOLS = 128
XROWS_CHUNK = 4
CHUNK = XROWS_CHUNK * XCOLS
LANES = 16
N_SLICES = CHUNK // LANES


def _sc_body(x_hbm, w_hbm, delta_hbm, out_hbm,
             idx0, idx1, rows0, rows1, dflat_v,
             gsem0, gsem1, wsem, isem):
    info = plsc.get_sparse_core_info()
    nc = info.num_cores
    nw = nc * info.num_subcores
    per_w = TOTAL // nw
    n_chunks = per_w // CHUNK

    idxb = (idx0, idx1)
    rowsb = (rows0, rows1)
    gsem = (gsem0, gsem1)

    wid = lax.axis_index("s") * nc + lax.axis_index("c")
    xrow_base = wid * (per_w // XCOLS)

    pltpu.sync_copy(delta_hbm, dflat_v)

    lane = lax.iota(jnp.int32, LANES)

    def issue_idx(c, p):
        pltpu.async_copy(
            x_hbm.at[pl.ds(xrow_base + c * XROWS_CHUNK, XROWS_CHUNK)],
            idxb[p], isem)

    def drain_idx(p):
        pltpu.make_async_copy(
            x_hbm.at[pl.ds(0, XROWS_CHUNK)], idxb[p], isem).wait()

    def issue_gathers(p):
        for r in range(XROWS_CHUNK):
            pltpu.async_copy(
                w_hbm.at[idxb[p].at[r]],
                rowsb[p].at[pl.ds(r * XCOLS, XCOLS)],
                gsem[p])

    def drain_gathers(p):
        for r in range(XROWS_CHUNK):
            pltpu.make_async_copy(
                w_hbm.at[idxb[p].at[r]],
                rowsb[p].at[pl.ds(r * XCOLS, XCOLS)],
                gsem[p]).wait()

    def issue_write(c, p):
        pltpu.async_copy(
            rowsb[p],
            out_hbm.at[pl.ds((xrow_base + c * XROWS_CHUNK) * XCOLS, CHUNK),
                       pl.ds(0, DIM)],
            wsem)

    def drain_write(p):
        pltpu.make_async_copy(
            rowsb[p],
            out_hbm.at[pl.ds(0, CHUNK), pl.ds(0, DIM)],
            wsem).wait()

    def fixup(p):
        def fix_slice(s, _):
            idxs = idxb[p][s // (XCOLS // LANES),
                           pl.ds((s % (XCOLS // LANES)) * LANES, LANES)]
            rel = idxs - BAND_LO
            m = (rel >= 0) & (rel < N_TRAIN)

            @pl.when(jnp.any(m))
            def _():
                relc = jnp.where(m, rel, 0)
                pos = s * LANES + lane

                def fix_col(col, _):
                    dvals = plsc.load_gather(
                        dflat_v, [col * N_TRAIN + relc], mask=m)
                    plsc.addupdate_scatter(
                        rowsb[p],
                        [pos, jnp.full((LANES,), 1, jnp.int32) * col],
                        dvals,
                        mask=m,
                    )
                    return _

                lax.fori_loop(0, DIM, fix_col, None, unroll=False)
            return _

        lax.fori_loop(0, N_SLICES, fix_slice, None, unroll=False)

    pltpu.sync_copy(x_hbm.at[pl.ds(xrow_base, XROWS_CHUNK)], idx0)
    issue_gathers(0)
    issue_idx(1, 1)

    def pair_body(g, _):
        for p in range(2):
            c = g * 2 + p
            q = 1 - p

            @pl.when(c < n_chunks - 1)
            def _():
                drain_idx(q)

            @pl.when(c >= 1)
            def _():
                drain_write(q)

            @pl.when(c < n_chunks - 1)
            def _():
                issue_gathers(q)

            drain_gathers(p)
            fixup(p)

            @pl.when(c < n_chunks - 2)
            def _():
                issue_idx(c + 2, p)

            issue_write(c, p)
        return _

    lax.fori_loop(0, n_chunks // 2, pair_body, None, unroll=False)
    drain_write((n_chunks - 1) % 2)


def kernel(x, W, delta):
    x_i = x if x.dtype == jnp.int32 else x.astype(jnp.int32)
    x2d = x_i.reshape(TOTAL // XCOLS, XCOLS)
    mesh = plsc.VectorSubcoreMesh(core_axis_name="c", subcore_axis_name="s")
    run = pl.kernel(
        _sc_body,
        out_type=jax.ShapeDtypeStruct((TOTAL, OUT_W), jnp.float32),
        mesh=mesh,
        scratch_types=[
            pltpu.VMEM((XROWS_CHUNK, XCOLS), jnp.int32),
            pltpu.VMEM((XROWS_CHUNK, XCOLS), jnp.int32),
            pltpu.VMEM((CHUNK, DIM), jnp.float32),
            pltpu.VMEM((CHUNK, DIM), jnp.float32),
            pltpu.VMEM((N_TRAIN * DIM,), jnp.float32),
            pltpu.SemaphoreType.DMA,
            pltpu.SemaphoreType.DMA,
            pltpu.SemaphoreType.DMA,
            pltpu.SemaphoreType.DMA,
        ],
        compiler_params=pltpu.CompilerParams(
            needs_layout_passes=False, use_tc_tiling_on_sc=False),
    )
    out = run(x2d, W, delta)
    return out.reshape(BATCH, HIST, OUT_W)[:, :, :DIM]
